# baseline (speedup 1.0000x reference)
import jax
import jax.numpy as jnp
from jax.experimental import pallas as pl
from jax.experimental.pallas import tpu as pltpu

L = 2048
DIM = 768
DINNER = 1536
DSTATE = 16
DCONV = 4
DTRANK = 48
E = 8
HID = 3072
HEADS = 12
HD = 64

ROWT = 256
MOET = 128

_CT = (((1,), (1,)), ((), ()))


def _dotT(a, b):
    return jax.lax.dot_general(a, b, _CT, preferred_element_type=jnp.float32)


def _silu(v):
    return v * jax.nn.sigmoid(v)


def _softplus(v):
    return jnp.where(v > 20.0, v, jnp.log1p(jnp.exp(jnp.minimum(v, 20.0))))


def _gelu(v):
    return 0.5 * v * (1.0 + jax.lax.erf(v * 0.7071067811865476))


def _mm_kernel(x_ref, w_ref, o_ref):
    o_ref[...] = _dotT(x_ref[...], w_ref[...])


def _matmul_T(x, w):
    n = w.shape[0]
    return pl.pallas_call(
        _mm_kernel,
        grid=(L // ROWT,),
        in_specs=[
            pl.BlockSpec((ROWT, x.shape[1]), lambda i: (i, 0)),
            pl.BlockSpec(w.shape, lambda i: (0, 0)),
        ],
        out_specs=pl.BlockSpec((ROWT, n), lambda i: (i, 0)),
        out_shape=jax.ShapeDtypeStruct((L, n), jnp.float32),
    )(x, w)


def _conv_delta_kernel(xc_ref, xp_ref, cwT_ref, cb_ref, xpd_ref, xpb_ref,
                       xpc_ref, dtw_ref, dtb_ref,
                       u_ref, dl_ref, bm_ref, cm_ref):
    i = pl.program_id(0)
    tile = xc_ref[...]
    prev3 = xp_ref[ROWT - 3:, :]
    prev3 = jnp.where(i == 0, 0.0, prev3)
    ext = jnp.concatenate([prev3, tile], axis=0)
    conv = cb_ref[...]
    for k in range(DCONV):
        conv = conv + cwT_ref[k:k + 1, :] * ext[k:k + ROWT, :]
    u = _silu(conv)
    u_ref[...] = u
    dtin = _dotT(u, xpd_ref[...])
    dl_ref[...] = _softplus(_dotT(dtin, dtw_ref[...]) + dtb_ref[...])
    bm_ref[...] = _dotT(u, xpb_ref[...])
    cm_ref[...] = _dotT(u, xpc_ref[...])


def _scan_kernel(u_ref, dl_ref, bm_ref, cm_ref, alogT_ref, d_ref, y_ref):
    AT = -jnp.exp(alogT_ref[...])
    D_row = d_ref[...]
    ri = jax.lax.broadcasted_iota(jnp.int32, (DSTATE, DSTATE), 0)
    ci = jax.lax.broadcasted_iota(jnp.int32, (DSTATE, DSTATE), 1)
    eye = jnp.where(ri == ci, 1.0, 0.0)

    def body(t, h):
        d_row = dl_ref[pl.ds(t, 1), :]
        u_row = u_ref[pl.ds(t, 1), :]
        dA = jnp.exp(AT * d_row)
        b_col = jnp.sum(eye * bm_ref[pl.ds(t, 1), :], axis=1, keepdims=True)
        c_col = jnp.sum(eye * cm_ref[pl.ds(t, 1), :], axis=1, keepdims=True)
        h = dA * h + (d_row * u_row) * b_col
        y = jnp.sum(h * c_col, axis=0, keepdims=True)
        y_ref[pl.ds(t, 1), :] = y + u_row * D_row
        return h

    jax.lax.fori_loop(0, L, body, jnp.zeros((DSTATE, DINNER), jnp.float32))


def _mamba_out_kernel(y_ref, res_ref, w_ref, x_ref, o_ref):
    y = y_ref[...] * _silu(res_ref[...])
    o_ref[...] = x_ref[...] + _dotT(y, w_ref[...])


def _mamba(x2d, in_proj_w, conv_w, conv_b, x_proj_w, dt_proj_w, dt_proj_b,
           A_log, D_param, out_proj_w):
    xr = _matmul_T(x2d, in_proj_w)

    cwT = jnp.transpose(conv_w)
    xpd = x_proj_w[:DTRANK]
    xpb = x_proj_w[DTRANK:DTRANK + DSTATE]
    xpc = x_proj_w[DTRANK + DSTATE:]
    full = lambda s: pl.BlockSpec(s, lambda i: (0, 0))
    u, dl, bm, cm = pl.pallas_call(
        _conv_delta_kernel,
        grid=(L // ROWT,),
        in_specs=[
            pl.BlockSpec((ROWT, DINNER), lambda i: (i, 0)),
            pl.BlockSpec((ROWT, DINNER), lambda i: (jnp.maximum(i - 1, 0), 0)),
            full((DCONV, DINNER)),
            full((1, DINNER)),
            full((DTRANK, DINNER)),
            full((DSTATE, DINNER)),
            full((DSTATE, DINNER)),
            full((DINNER, DTRANK)),
            full((1, DINNER)),
        ],
        out_specs=[
            pl.BlockSpec((ROWT, DINNER), lambda i: (i, 0)),
            pl.BlockSpec((ROWT, DINNER), lambda i: (i, 0)),
            pl.BlockSpec((ROWT, DSTATE), lambda i: (i, 0)),
            pl.BlockSpec((ROWT, DSTATE), lambda i: (i, 0)),
        ],
        out_shape=[
            jax.ShapeDtypeStruct((L, DINNER), jnp.float32),
            jax.ShapeDtypeStruct((L, DINNER), jnp.float32),
            jax.ShapeDtypeStruct((L, DSTATE), jnp.float32),
            jax.ShapeDtypeStruct((L, DSTATE), jnp.float32),
        ],
    )(xr[:, :DINNER], xr[:, :DINNER], cwT, conv_b.reshape(1, DINNER),
      xpd, xpb, xpc, dt_proj_w, dt_proj_b.reshape(1, DINNER))

    ycore = pl.pallas_call(
        _scan_kernel,
        in_specs=[pl.BlockSpec((L, DINNER), lambda: (0, 0)),
                  pl.BlockSpec((L, DINNER), lambda: (0, 0)),
                  pl.BlockSpec((L, DSTATE), lambda: (0, 0)),
                  pl.BlockSpec((L, DSTATE), lambda: (0, 0)),
                  pl.BlockSpec((DSTATE, DINNER), lambda: (0, 0)),
                  pl.BlockSpec((1, DINNER), lambda: (0, 0))],
        out_specs=pl.BlockSpec((L, DINNER), lambda: (0, 0)),
        out_shape=jax.ShapeDtypeStruct((L, DINNER), jnp.float32),
    )(u, dl, bm, cm, jnp.transpose(A_log), D_param.reshape(1, DINNER))

    return pl.pallas_call(
        _mamba_out_kernel,
        grid=(L // ROWT,),
        in_specs=[
            pl.BlockSpec((ROWT, DINNER), lambda i: (i, 0)),
            pl.BlockSpec((ROWT, DINNER), lambda i: (i, 1)),
            pl.BlockSpec((DIM, DINNER), lambda i: (0, 0)),
            pl.BlockSpec((ROWT, DIM), lambda i: (i, 0)),
        ],
        out_specs=pl.BlockSpec((ROWT, DIM), lambda i: (i, 0)),
        out_shape=jax.ShapeDtypeStruct((L, DIM), jnp.float32),
    )(ycore, xr, out_proj_w, x2d)


def _router_kernel(x_ref, gw_ref, gb_ref, pos_ref, scale_ref, se_ref):
    logits = _dotT(x_ref[...], gw_ref[...]) + gb_ref[...]
    m = jnp.max(logits, axis=1, keepdims=True)
    ex = jnp.exp(logits - m)
    gs = ex / jnp.sum(ex, axis=1, keepdims=True)
    p = jnp.max(gs, axis=1, keepdims=True)
    lane = jax.lax.broadcasted_iota(jnp.int32, (L, E), 1)
    cand = jnp.where(gs >= p, lane, jnp.int32(E))
    eid = jnp.min(cand, axis=1, keepdims=True)
    oh = jnp.where(lane == eid, 1.0, 0.0)

    rt = jax.lax.broadcasted_iota(jnp.int32, (L, L), 0)
    ct = jax.lax.broadcasted_iota(jnp.int32, (L, L), 1)
    strict = jnp.where(rt > ct, 1.0, 0.0)
    rank = jax.lax.dot_general(strict, oh, (((1,), (0,)), ((), ())),
                               precision=jax.lax.Precision.HIGHEST,
                               preferred_element_type=jnp.float32)
    counts = jnp.sum(oh, axis=0, keepdims=True)
    re = jax.lax.broadcasted_iota(jnp.int32, (E, E), 0)
    ce = jax.lax.broadcasted_iota(jnp.int32, (E, E), 1)
    upper = jnp.where(re < ce, 1.0, 0.0)
    offs = jax.lax.dot_general(counts, upper, (((1,), (0,)), ((), ())),
                               precision=jax.lax.Precision.HIGHEST,
                               preferred_element_type=jnp.float32)
    pos = jnp.sum(oh * (rank + offs), axis=1, keepdims=True)
    pos_ref[...] = pos.astype(jnp.int32)
    scale_ref[...] = p / (p + 1e-6)
    se = jnp.concatenate([offs, offs + counts], axis=0)
    se_ref[...] = se.astype(jnp.int32)


def _scatter_kernel(pos_ref, x_ref, xs_ref):
    def body(t, _):
        xs_ref[pl.ds(pos_ref[t], 1), :] = x_ref[pl.ds(t, 1), :]
        return 0
    jax.lax.fori_loop(0, L, body, 0)


def _expert_kernel(se_ref, xs_ref, w1_ref, b1_ref, w2_ref, b2_ref, o_ref):
    e = pl.program_id(0)

    @pl.when(e == 0)
    def _():
        o_ref[...] = jnp.zeros_like(o_ref)

    start = se_ref[0, e]
    end = se_ref[1, e]
    t0 = start // MOET
    t1 = (end + MOET - 1) // MOET

    def body(i, _):
        r0 = i * MOET
        rows = xs_ref[pl.ds(r0, MOET), :]
        h = _gelu(_dotT(rows, w1_ref[0]) + b1_ref[0])
        o = _dotT(h, w2_ref[0]) + b2_ref[0]
        ids = r0 + jax.lax.broadcasted_iota(jnp.int32, (MOET, 1), 0)
        mask = jnp.logical_and(ids >= start, ids < end)
        o_ref[pl.ds(r0, MOET), :] += jnp.where(mask, o, 0.0)
        return 0

    jax.lax.fori_loop(t0, t1, body, 0)


def _gather_kernel(pos_ref, os_ref, x_ref, scale_ref, o_ref):
    def body(t, _):
        o_ref[pl.ds(t, 1), :] = (
            x_ref[pl.ds(t, 1), :]
            + scale_ref[pl.ds(t, 1), :] * os_ref[pl.ds(pos_ref[t], 1), :])
        return 0
    jax.lax.fori_loop(0, L, body, 0)


def _moe(x2d, gate_w, gate_b, e_w1, e_b1, e_w2, e_b2):
    pos, scale, se = pl.pallas_call(
        _router_kernel,
        in_specs=[pl.BlockSpec((L, DIM), lambda: (0, 0)),
                  pl.BlockSpec((E, DIM), lambda: (0, 0)),
                  pl.BlockSpec((1, E), lambda: (0, 0))],
        out_specs=[pl.BlockSpec((L, 1), lambda: (0, 0)),
                   pl.BlockSpec((L, 1), lambda: (0, 0)),
                   pl.BlockSpec((2, E), lambda: (0, 0))],
        out_shape=[jax.ShapeDtypeStruct((L, 1), jnp.int32),
                   jax.ShapeDtypeStruct((L, 1), jnp.float32),
                   jax.ShapeDtypeStruct((2, E), jnp.int32)],
    )(x2d, gate_w, gate_b.reshape(1, E))

    pos1 = pos.reshape(L)
    xs = pl.pallas_call(
        _scatter_kernel,
        grid_spec=pltpu.PrefetchScalarGridSpec(
            num_scalar_prefetch=1,
            grid=(1,),
            in_specs=[pl.BlockSpec((L, DIM), lambda i, p: (0, 0))],
            out_specs=pl.BlockSpec((L, DIM), lambda i, p: (0, 0)),
        ),
        out_shape=jax.ShapeDtypeStruct((L, DIM), jnp.float32),
    )(pos1, x2d)

    outs = pl.pallas_call(
        _expert_kernel,
        grid_spec=pltpu.PrefetchScalarGridSpec(
            num_scalar_prefetch=1,
            grid=(E,),
            in_specs=[
                pl.BlockSpec((L, DIM), lambda e, s: (0, 0)),
                pl.BlockSpec((1, HID, DIM), lambda e, s: (e, 0, 0)),
                pl.BlockSpec((1, 1, HID), lambda e, s: (e, 0, 0)),
                pl.BlockSpec((1, DIM, HID), lambda e, s: (e, 0, 0)),
                pl.BlockSpec((1, 1, DIM), lambda e, s: (e, 0, 0)),
            ],
            out_specs=pl.BlockSpec((L, DIM), lambda e, s: (0, 0)),
        ),
        out_shape=jax.ShapeDtypeStruct((L, DIM), jnp.float32),
    )(se, xs, e_w1, e_b1.reshape(E, 1, HID), e_w2, e_b2.reshape(E, 1, DIM))

    return pl.pallas_call(
        _gather_kernel,
        grid_spec=pltpu.PrefetchScalarGridSpec(
            num_scalar_prefetch=1,
            grid=(1,),
            in_specs=[pl.BlockSpec((L, DIM), lambda i, p: (0, 0)),
                      pl.BlockSpec((L, DIM), lambda i, p: (0, 0)),
                      pl.BlockSpec((L, 1), lambda i, p: (0, 0))],
            out_specs=pl.BlockSpec((L, DIM), lambda i, p: (0, 0)),
        ),
        out_shape=jax.ShapeDtypeStruct((L, DIM), jnp.float32),
    )(pos1, outs, x2d, scale)


def _gi_kernel(x_ref, w_ref, b_ref, o_ref):
    o_ref[...] = _dotT(x_ref[...], w_ref[...]) + b_ref[...]


def _gru_seq_kernel(gi_ref, whhT_ref, bhh_ref, x_ref, o_ref):
    whhT = whhT_ref[...]
    bhh = bhh_ref[...]

    def body(t, h):
        gh = jnp.dot(h.astype(jnp.bfloat16), whhT,
                     preferred_element_type=jnp.float32) + bhh
        gi = gi_ref[pl.ds(t, 1), :]
        r = jax.nn.sigmoid(gi[:, :DIM] + gh[:, :DIM])
        z = jax.nn.sigmoid(gi[:, DIM:2 * DIM] + gh[:, DIM:2 * DIM])
        n = jnp.tanh(gi[:, 2 * DIM:] + r * gh[:, 2 * DIM:])
        h = (1.0 - z) * n + z * h
        o_ref[pl.ds(t, 1), :] = x_ref[pl.ds(t, 1), :] + h
        return h

    jax.lax.fori_loop(0, L, body, jnp.zeros((1, DIM), jnp.float32),
                      unroll=8)


def _gru(x2d, w_ih, w_hh, b_ih, b_hh):
    gi = pl.pallas_call(
        _gi_kernel,
        grid=(L // ROWT,),
        in_specs=[pl.BlockSpec((ROWT, DIM), lambda i: (i, 0)),
                  pl.BlockSpec((3 * DIM, DIM), lambda i: (0, 0)),
                  pl.BlockSpec((1, 3 * DIM), lambda i: (0, 0))],
        out_specs=pl.BlockSpec((ROWT, 3 * DIM), lambda i: (i, 0)),
        out_shape=jax.ShapeDtypeStruct((L, 3 * DIM), jnp.float32),
    )(x2d, w_ih, b_ih.reshape(1, 3 * DIM))

    return pl.pallas_call(
        _gru_seq_kernel,
        in_specs=[pl.BlockSpec((L, 3 * DIM), lambda: (0, 0)),
                  pl.BlockSpec((DIM, 3 * DIM), lambda: (0, 0)),
                  pl.BlockSpec((1, 3 * DIM), lambda: (0, 0)),
                  pl.BlockSpec((L, DIM), lambda: (0, 0))],
        out_specs=pl.BlockSpec((L, DIM), lambda: (0, 0)),
        out_shape=jax.ShapeDtypeStruct((L, DIM), jnp.float32),
    )(gi, jnp.transpose(w_hh).astype(jnp.bfloat16),
      b_hh.reshape(1, 3 * DIM), x2d)


def _attn_head_kernel(q_ref, k_ref, v_ref, o_ref):
    att = _dotT(q_ref[0], k_ref[0]) * 0.125
    m = jnp.max(att, axis=1, keepdims=True)
    ex = jnp.exp(att - m)
    s = jnp.sum(ex, axis=1, keepdims=True)
    o_ref[0] = jnp.dot(ex, v_ref[0], preferred_element_type=jnp.float32) / s


def _attn_out_kernel(o_ref, w_ref, b_ref, x_ref, out_ref):
    out_ref[...] = x_ref[...] + _dotT(o_ref[...], w_ref[...]) + b_ref[...]


def _mha(x2d, in_w, in_b, out_w, out_b):
    qkv = pl.pallas_call(
        _gi_kernel,
        grid=(L // ROWT,),
        in_specs=[pl.BlockSpec((ROWT, DIM), lambda i: (i, 0)),
                  pl.BlockSpec((3 * DIM, DIM), lambda i: (0, 0)),
                  pl.BlockSpec((1, 3 * DIM), lambda i: (0, 0))],
        out_specs=pl.BlockSpec((ROWT, 3 * DIM), lambda i: (i, 0)),
        out_shape=jax.ShapeDtypeStruct((L, 3 * DIM), jnp.float32),
    )(x2d, in_w, in_b.reshape(1, 3 * DIM))

    qkv_h = jnp.transpose(qkv.reshape(L, 3 * HEADS, HD), (1, 0, 2))
    o = pl.pallas_call(
        _attn_head_kernel,
        grid=(HEADS,),
        in_specs=[pl.BlockSpec((1, L, HD), lambda h: (h, 0, 0)),
                  pl.BlockSpec((1, L, HD), lambda h: (HEADS + h, 0, 0)),
                  pl.BlockSpec((1, L, HD), lambda h: (2 * HEADS + h, 0, 0))],
        out_specs=pl.BlockSpec((1, L, HD), lambda h: (h, 0, 0)),
        out_shape=jax.ShapeDtypeStruct((HEADS, L, HD), jnp.float32),
    )(qkv_h, qkv_h, qkv_h)
    o = jnp.transpose(o, (1, 0, 2)).reshape(L, DIM)

    return pl.pallas_call(
        _attn_out_kernel,
        grid=(L // ROWT,),
        in_specs=[pl.BlockSpec((ROWT, DIM), lambda i: (i, 0)),
                  pl.BlockSpec((DIM, DIM), lambda i: (0, 0)),
                  pl.BlockSpec((1, DIM), lambda i: (0, 0)),
                  pl.BlockSpec((ROWT, DIM), lambda i: (i, 0))],
        out_specs=pl.BlockSpec((ROWT, DIM), lambda i: (i, 0)),
        out_shape=jax.ShapeDtypeStruct((L, DIM), jnp.float32),
    )(o, out_w, out_b.reshape(1, DIM), x2d)


def kernel(x, in_proj_w, conv_w, conv_b, x_proj_w, dt_proj_w, dt_proj_b,
           A_log, D_param, out_proj_w, gate_w, gate_b, e_w1, e_b1, e_w2,
           e_b2, gru_w_ih, gru_w_hh, gru_b_ih, gru_b_hh, attn_in_w,
           attn_in_b, attn_out_w, attn_out_b):
    x2d = x.reshape(L, DIM)
    x1 = _mamba(x2d, in_proj_w, conv_w, conv_b, x_proj_w, dt_proj_w,
                dt_proj_b, A_log, D_param, out_proj_w)
    x2 = _moe(x1, gate_w, gate_b, e_w1, e_b1, e_w2, e_b2)
    x3 = _gru(x2, gru_w_ih, gru_w_hh, gru_b_ih, gru_b_hh)
    x4 = _mha(x3, attn_in_w, attn_in_b, attn_out_w, attn_out_b)
    return x4.reshape(1, L, DIM)


# speedup vs baseline: 1.0745x; 1.0745x over previous
import jax
import jax.numpy as jnp
from jax.experimental import pallas as pl
from jax.experimental.pallas import tpu as pltpu
from jax.experimental.pallas import tpu_sc as plsc
import functools

L = 2048
DIM = 768
DINNER = 1536
DSTATE = 16
DCONV = 4
DTRANK = 48
E = 8
HID = 3072
HEADS = 12
HD = 64

ROWT = 256
MOET = 128

_CT = (((1,), (1,)), ((), ()))


def _dotT(a, b):
    return jax.lax.dot_general(a, b, _CT, preferred_element_type=jnp.float32)


def _silu(v):
    return v * jax.nn.sigmoid(v)


def _softplus(v):
    return jnp.where(v > 20.0, v, jnp.log1p(jnp.exp(jnp.minimum(v, 20.0))))


def _gelu(v):
    return 0.5 * v * (1.0 + jax.lax.erf(v * 0.7071067811865476))


def _mm_kernel(x_ref, w_ref, o_ref):
    o_ref[...] = _dotT(x_ref[...], w_ref[...])


def _matmul_T(x, w):
    n = w.shape[0]
    return pl.pallas_call(
        _mm_kernel,
        grid=(L // ROWT,),
        in_specs=[
            pl.BlockSpec((ROWT, x.shape[1]), lambda i: (i, 0)),
            pl.BlockSpec(w.shape, lambda i: (0, 0)),
        ],
        out_specs=pl.BlockSpec((ROWT, n), lambda i: (i, 0)),
        out_shape=jax.ShapeDtypeStruct((L, n), jnp.float32),
    )(x, w)


def _conv_delta_kernel(xc_ref, xp_ref, cwT_ref, cb_ref, xpd_ref, xpb_ref,
                       xpc_ref, dtw_ref, dtb_ref,
                       u_ref, dl_ref, bm_ref, cm_ref):
    i = pl.program_id(0)
    tile = xc_ref[...]
    prev3 = xp_ref[ROWT - 3:, :]
    prev3 = jnp.where(i == 0, 0.0, prev3)
    ext = jnp.concatenate([prev3, tile], axis=0)
    conv = cb_ref[...]
    for k in range(DCONV):
        conv = conv + cwT_ref[k:k + 1, :] * ext[k:k + ROWT, :]
    u = _silu(conv)
    u_ref[...] = u
    dtin = _dotT(u, xpd_ref[...])
    dl_ref[...] = _softplus(_dotT(dtin, dtw_ref[...]) + dtb_ref[...])
    bm_ref[...] = _dotT(u, xpb_ref[...])
    cm_ref[...] = _dotT(u, xpc_ref[...])


def _scan_kernel(u_ref, dl_ref, bm_ref, cm_ref, alogT_ref, d_ref, y_ref):
    AT = -jnp.exp(alogT_ref[...])
    D_row = d_ref[...]
    ri = jax.lax.broadcasted_iota(jnp.int32, (DSTATE, DSTATE), 0)
    ci = jax.lax.broadcasted_iota(jnp.int32, (DSTATE, DSTATE), 1)
    eye = jnp.where(ri == ci, 1.0, 0.0)

    def body(t, h):
        d_row = dl_ref[pl.ds(t, 1), :]
        u_row = u_ref[pl.ds(t, 1), :]
        dA = jnp.exp(AT * d_row)
        b_col = jnp.sum(eye * bm_ref[pl.ds(t, 1), :], axis=1, keepdims=True)
        c_col = jnp.sum(eye * cm_ref[pl.ds(t, 1), :], axis=1, keepdims=True)
        h = dA * h + (d_row * u_row) * b_col
        y = jnp.sum(h * c_col, axis=0, keepdims=True)
        y_ref[pl.ds(t, 1), :] = y + u_row * D_row
        return h

    jax.lax.fori_loop(0, L, body, jnp.zeros((DSTATE, DINNER), jnp.float32))


def _mamba_out_kernel(y_ref, res_ref, w_ref, x_ref, o_ref):
    y = y_ref[...] * _silu(res_ref[...])
    o_ref[...] = x_ref[...] + _dotT(y, w_ref[...])


def _mamba(x2d, in_proj_w, conv_w, conv_b, x_proj_w, dt_proj_w, dt_proj_b,
           A_log, D_param, out_proj_w):
    xr = _matmul_T(x2d, in_proj_w)

    cwT = jnp.transpose(conv_w)
    xpd = x_proj_w[:DTRANK]
    xpb = x_proj_w[DTRANK:DTRANK + DSTATE]
    xpc = x_proj_w[DTRANK + DSTATE:]
    full = lambda s: pl.BlockSpec(s, lambda i: (0, 0))
    u, dl, bm, cm = pl.pallas_call(
        _conv_delta_kernel,
        grid=(L // ROWT,),
        in_specs=[
            pl.BlockSpec((ROWT, DINNER), lambda i: (i, 0)),
            pl.BlockSpec((ROWT, DINNER), lambda i: (jnp.maximum(i - 1, 0), 0)),
            full((DCONV, DINNER)),
            full((1, DINNER)),
            full((DTRANK, DINNER)),
            full((DSTATE, DINNER)),
            full((DSTATE, DINNER)),
            full((DINNER, DTRANK)),
            full((1, DINNER)),
        ],
        out_specs=[
            pl.BlockSpec((ROWT, DINNER), lambda i: (i, 0)),
            pl.BlockSpec((ROWT, DINNER), lambda i: (i, 0)),
            pl.BlockSpec((ROWT, DSTATE), lambda i: (i, 0)),
            pl.BlockSpec((ROWT, DSTATE), lambda i: (i, 0)),
        ],
        out_shape=[
            jax.ShapeDtypeStruct((L, DINNER), jnp.float32),
            jax.ShapeDtypeStruct((L, DINNER), jnp.float32),
            jax.ShapeDtypeStruct((L, DSTATE), jnp.float32),
            jax.ShapeDtypeStruct((L, DSTATE), jnp.float32),
        ],
    )(xr[:, :DINNER], xr[:, :DINNER], cwT, conv_b.reshape(1, DINNER),
      xpd, xpb, xpc, dt_proj_w, dt_proj_b.reshape(1, DINNER))

    ycore = pl.pallas_call(
        _scan_kernel,
        in_specs=[pl.BlockSpec((L, DINNER), lambda: (0, 0)),
                  pl.BlockSpec((L, DINNER), lambda: (0, 0)),
                  pl.BlockSpec((L, DSTATE), lambda: (0, 0)),
                  pl.BlockSpec((L, DSTATE), lambda: (0, 0)),
                  pl.BlockSpec((DSTATE, DINNER), lambda: (0, 0)),
                  pl.BlockSpec((1, DINNER), lambda: (0, 0))],
        out_specs=pl.BlockSpec((L, DINNER), lambda: (0, 0)),
        out_shape=jax.ShapeDtypeStruct((L, DINNER), jnp.float32),
    )(u, dl, bm, cm, jnp.transpose(A_log), D_param.reshape(1, DINNER))

    return pl.pallas_call(
        _mamba_out_kernel,
        grid=(L // ROWT,),
        in_specs=[
            pl.BlockSpec((ROWT, DINNER), lambda i: (i, 0)),
            pl.BlockSpec((ROWT, DINNER), lambda i: (i, 1)),
            pl.BlockSpec((DIM, DINNER), lambda i: (0, 0)),
            pl.BlockSpec((ROWT, DIM), lambda i: (i, 0)),
        ],
        out_specs=pl.BlockSpec((ROWT, DIM), lambda i: (i, 0)),
        out_shape=jax.ShapeDtypeStruct((L, DIM), jnp.float32),
    )(ycore, xr, out_proj_w, x2d)


def _router_kernel(x_ref, gw_ref, gb_ref, pos_ref, se_ref, perm_ref, scs_ref):
    logits = _dotT(x_ref[...], gw_ref[...]) + gb_ref[...]
    m = jnp.max(logits, axis=1, keepdims=True)
    ex = jnp.exp(logits - m)
    gs = ex / jnp.sum(ex, axis=1, keepdims=True)
    p = jnp.max(gs, axis=1, keepdims=True)
    lane = jax.lax.broadcasted_iota(jnp.int32, (L, E), 1)
    cand = jnp.where(gs >= p, lane, jnp.int32(E))
    eid = jnp.min(cand, axis=1, keepdims=True)
    oh = jnp.where(lane == eid, 1.0, 0.0)

    rt = jax.lax.broadcasted_iota(jnp.int32, (L, L), 0)
    ct = jax.lax.broadcasted_iota(jnp.int32, (L, L), 1)
    strict = jnp.where(rt > ct, 1.0, 0.0)
    rank = jax.lax.dot_general(strict, oh, (((1,), (0,)), ((), ())),
                               precision=jax.lax.Precision.HIGHEST,
                               preferred_element_type=jnp.float32)
    counts = jnp.sum(oh, axis=0, keepdims=True)
    re = jax.lax.broadcasted_iota(jnp.int32, (E, E), 0)
    ce = jax.lax.broadcasted_iota(jnp.int32, (E, E), 1)
    upper = jnp.where(re < ce, 1.0, 0.0)
    offs = jax.lax.dot_general(counts, upper, (((1,), (0,)), ((), ())),
                               precision=jax.lax.Precision.HIGHEST,
                               preferred_element_type=jnp.float32)
    pos = jnp.sum(oh * (rank + offs), axis=1, keepdims=True)
    pos_ref[...] = pos.astype(jnp.int32)
    se = jnp.concatenate([offs, offs + counts], axis=0)
    se_ref[...] = se.astype(jnp.int32)
    slot = jax.lax.broadcasted_iota(jnp.int32, (L, L), 1)
    psel = jnp.where(slot == pos.astype(jnp.int32), 1.0, 0.0)
    tcol = jax.lax.broadcasted_iota(jnp.int32, (L, 1), 0).astype(jnp.float32)
    perm_ref[...] = jax.lax.dot_general(
        tcol, psel, (((0,), (0,)), ((), ())),
        precision=jax.lax.Precision.HIGHEST,
        preferred_element_type=jnp.float32).astype(jnp.int32)
    scale = p / (p + 1e-6)
    scs_ref[...] = jax.lax.dot_general(
        scale, psel, (((0,), (0,)), ((), ())),
        precision=jax.lax.Precision.HIGHEST,
        preferred_element_type=jnp.float32)


def _sc_gather_body(nw, bpw, d, table_hbm, idx_hbm, out_hbm, idx_v, rows_v,
                    sem):
    wid = jax.lax.axis_index("s") * 2 + jax.lax.axis_index("c")
    base = wid * bpw
    pltpu.sync_copy(idx_hbm.at[pl.ds(base, bpw)], idx_v)
    pltpu.async_copy(table_hbm.at[idx_v], rows_v, sem).wait()
    pltpu.sync_copy(rows_v, out_hbm.at[pl.ds(base, bpw)])


def _sc_gather(table, idx):
    b = idx.shape[0]
    d = table.shape[1]
    nw = 32
    bpw = b // nw
    mesh = plsc.VectorSubcoreMesh(core_axis_name="c", subcore_axis_name="s")
    k = functools.partial(
        pl.kernel,
        mesh=mesh,
        out_type=jax.ShapeDtypeStruct((b, d), jnp.float32),
        scratch_types=[pltpu.VMEM((bpw,), jnp.int32),
                       pltpu.VMEM((bpw, d), jnp.float32),
                       pltpu.SemaphoreType.DMA],
    )(functools.partial(_sc_gather_body, nw, bpw, d))
    return k(table, idx)


def _expert_kernel(se_ref, xs_ref, scs_ref, w1_ref, b1_ref, w2_ref, b2_ref, o_ref):
    e = pl.program_id(0)

    @pl.when(e == 0)
    def _():
        o_ref[...] = jnp.zeros_like(o_ref)

    start = se_ref[0, e]
    end = se_ref[1, e]
    t0 = start // MOET
    t1 = (end + MOET - 1) // MOET

    def body(i, _):
        r0 = i * MOET
        rows = xs_ref[pl.ds(r0, MOET), :]
        h = _gelu(_dotT(rows, w1_ref[0]) + b1_ref[0])
        o = (_dotT(h, w2_ref[0]) + b2_ref[0]) * scs_ref[pl.ds(r0, MOET), :]
        ids = r0 + jax.lax.broadcasted_iota(jnp.int32, (MOET, 1), 0)
        mask = jnp.logical_and(ids >= start, ids < end)
        o_ref[pl.ds(r0, MOET), :] += jnp.where(mask, o, 0.0)
        return 0

    jax.lax.fori_loop(t0, t1, body, 0)


def _moe(x2d, gate_w, gate_b, e_w1, e_b1, e_w2, e_b2):
    pos, se, perm, scs = pl.pallas_call(
        _router_kernel,
        in_specs=[pl.BlockSpec((L, DIM), lambda: (0, 0)),
                  pl.BlockSpec((E, DIM), lambda: (0, 0)),
                  pl.BlockSpec((1, E), lambda: (0, 0))],
        out_specs=[pl.BlockSpec((L, 1), lambda: (0, 0)),
                   pl.BlockSpec((2, E), lambda: (0, 0)),
                   pl.BlockSpec((1, L), lambda: (0, 0)),
                   pl.BlockSpec((1, L), lambda: (0, 0))],
        out_shape=[jax.ShapeDtypeStruct((L, 1), jnp.int32),
                   jax.ShapeDtypeStruct((2, E), jnp.int32),
                   jax.ShapeDtypeStruct((1, L), jnp.int32),
                   jax.ShapeDtypeStruct((1, L), jnp.float32)],
    )(x2d, gate_w, gate_b.reshape(1, E))

    xs = _sc_gather(x2d, perm.reshape(L))

    outs = pl.pallas_call(
        _expert_kernel,
        grid_spec=pltpu.PrefetchScalarGridSpec(
            num_scalar_prefetch=1,
            grid=(E,),
            in_specs=[
                pl.BlockSpec((L, DIM), lambda e, s: (0, 0)),
                pl.BlockSpec((L, 1), lambda e, s: (0, 0)),
                pl.BlockSpec((1, HID, DIM), lambda e, s: (e, 0, 0)),
                pl.BlockSpec((1, 1, HID), lambda e, s: (e, 0, 0)),
                pl.BlockSpec((1, DIM, HID), lambda e, s: (e, 0, 0)),
                pl.BlockSpec((1, 1, DIM), lambda e, s: (e, 0, 0)),
            ],
            out_specs=pl.BlockSpec((L, DIM), lambda e, s: (0, 0)),
        ),
        out_shape=jax.ShapeDtypeStruct((L, DIM), jnp.float32),
    )(se, xs, scs.reshape(L, 1), e_w1, e_b1.reshape(E, 1, HID), e_w2,
      e_b2.reshape(E, 1, DIM))

    return _sc_gather(outs, pos.reshape(L))


def _gi_kernel(x_ref, w_ref, b_ref, o_ref):
    o_ref[...] = _dotT(x_ref[...], w_ref[...]) + b_ref[...]


def _gi_resid_kernel(x_ref, m_ref, w_ref, b_ref, x2_ref, o_ref):
    x2 = x_ref[...] + m_ref[...]
    x2_ref[...] = x2
    o_ref[...] = _dotT(x2, w_ref[...]) + b_ref[...]


def _gru_seq_kernel(gi_ref, whhT_ref, bhh_ref, x_ref, o_ref):
    whhT = whhT_ref[...]
    bhh = bhh_ref[...]

    def body(t, h):
        gh = jnp.dot(h.astype(jnp.bfloat16), whhT,
                     preferred_element_type=jnp.float32) + bhh
        gi = gi_ref[pl.ds(t, 1), :]
        r = jax.nn.sigmoid(gi[:, :DIM] + gh[:, :DIM])
        z = jax.nn.sigmoid(gi[:, DIM:2 * DIM] + gh[:, DIM:2 * DIM])
        n = jnp.tanh(gi[:, 2 * DIM:] + r * gh[:, 2 * DIM:])
        h = (1.0 - z) * n + z * h
        o_ref[pl.ds(t, 1), :] = x_ref[pl.ds(t, 1), :] + h
        return h

    jax.lax.fori_loop(0, L, body, jnp.zeros((1, DIM), jnp.float32))


def _gru(x1, moe, w_ih, w_hh, b_ih, b_hh):
    x2d, gi = pl.pallas_call(
        _gi_resid_kernel,
        grid=(L // ROWT,),
        in_specs=[pl.BlockSpec((ROWT, DIM), lambda i: (i, 0)),
                  pl.BlockSpec((ROWT, DIM), lambda i: (i, 0)),
                  pl.BlockSpec((3 * DIM, DIM), lambda i: (0, 0)),
                  pl.BlockSpec((1, 3 * DIM), lambda i: (0, 0))],
        out_specs=[pl.BlockSpec((ROWT, DIM), lambda i: (i, 0)),
                   pl.BlockSpec((ROWT, 3 * DIM), lambda i: (i, 0))],
        out_shape=[jax.ShapeDtypeStruct((L, DIM), jnp.float32),
                   jax.ShapeDtypeStruct((L, 3 * DIM), jnp.float32)],
    )(x1, moe, w_ih, b_ih.reshape(1, 3 * DIM))

    return pl.pallas_call(
        _gru_seq_kernel,
        in_specs=[pl.BlockSpec((L, 3 * DIM), lambda: (0, 0)),
                  pl.BlockSpec((DIM, 3 * DIM), lambda: (0, 0)),
                  pl.BlockSpec((1, 3 * DIM), lambda: (0, 0)),
                  pl.BlockSpec((L, DIM), lambda: (0, 0))],
        out_specs=pl.BlockSpec((L, DIM), lambda: (0, 0)),
        out_shape=jax.ShapeDtypeStruct((L, DIM), jnp.float32),
    )(gi, jnp.transpose(w_hh).astype(jnp.bfloat16),
      b_hh.reshape(1, 3 * DIM), x2d)


def _attn_head_kernel(q_ref, k_ref, v_ref, o_ref):
    att = _dotT(q_ref[0], k_ref[0]) * 0.125
    m = jnp.max(att, axis=1, keepdims=True)
    ex = jnp.exp(att - m)
    s = jnp.sum(ex, axis=1, keepdims=True)
    o_ref[0] = jnp.dot(ex, v_ref[0], preferred_element_type=jnp.float32) / s


def _attn_out_kernel(o_ref, w_ref, b_ref, x_ref, out_ref):
    out_ref[...] = x_ref[...] + _dotT(o_ref[...], w_ref[...]) + b_ref[...]


def _mha(x2d, in_w, in_b, out_w, out_b):
    qkv = pl.pallas_call(
        _gi_kernel,
        grid=(L // ROWT,),
        in_specs=[pl.BlockSpec((ROWT, DIM), lambda i: (i, 0)),
                  pl.BlockSpec((3 * DIM, DIM), lambda i: (0, 0)),
                  pl.BlockSpec((1, 3 * DIM), lambda i: (0, 0))],
        out_specs=pl.BlockSpec((ROWT, 3 * DIM), lambda i: (i, 0)),
        out_shape=jax.ShapeDtypeStruct((L, 3 * DIM), jnp.float32),
    )(x2d, in_w, in_b.reshape(1, 3 * DIM))

    qkv_h = jnp.transpose(qkv.reshape(L, 3 * HEADS, HD), (1, 0, 2))
    o = pl.pallas_call(
        _attn_head_kernel,
        grid=(HEADS,),
        in_specs=[pl.BlockSpec((1, L, HD), lambda h: (h, 0, 0)),
                  pl.BlockSpec((1, L, HD), lambda h: (HEADS + h, 0, 0)),
                  pl.BlockSpec((1, L, HD), lambda h: (2 * HEADS + h, 0, 0))],
        out_specs=pl.BlockSpec((1, L, HD), lambda h: (h, 0, 0)),
        out_shape=jax.ShapeDtypeStruct((HEADS, L, HD), jnp.float32),
    )(qkv_h, qkv_h, qkv_h)
    o = jnp.transpose(o, (1, 0, 2)).reshape(L, DIM)

    return pl.pallas_call(
        _attn_out_kernel,
        grid=(L // ROWT,),
        in_specs=[pl.BlockSpec((ROWT, DIM), lambda i: (i, 0)),
                  pl.BlockSpec((DIM, DIM), lambda i: (0, 0)),
                  pl.BlockSpec((1, DIM), lambda i: (0, 0)),
                  pl.BlockSpec((ROWT, DIM), lambda i: (i, 0))],
        out_specs=pl.BlockSpec((ROWT, DIM), lambda i: (i, 0)),
        out_shape=jax.ShapeDtypeStruct((L, DIM), jnp.float32),
    )(o, out_w, out_b.reshape(1, DIM), x2d)


def kernel(x, in_proj_w, conv_w, conv_b, x_proj_w, dt_proj_w, dt_proj_b,
           A_log, D_param, out_proj_w, gate_w, gate_b, e_w1, e_b1, e_w2,
           e_b2, gru_w_ih, gru_w_hh, gru_b_ih, gru_b_hh, attn_in_w,
           attn_in_b, attn_out_w, attn_out_b):
    x2d = x.reshape(L, DIM)
    x1 = _mamba(x2d, in_proj_w, conv_w, conv_b, x_proj_w, dt_proj_w,
                dt_proj_b, A_log, D_param, out_proj_w)
    moe = _moe(x1, gate_w, gate_b, e_w1, e_b1, e_w2, e_b2)
    x3 = _gru(x1, moe, gru_w_ih, gru_w_hh, gru_b_ih, gru_b_hh)
    x4 = _mha(x3, attn_in_w, attn_in_b, attn_out_w, attn_out_b)
    return x4.reshape(1, L, DIM)
